# Initial kernel scaffold; baseline (speedup 1.0000x reference)
import jax
import jax.numpy as jnp
from jax.experimental import pallas as pl


def kernel(Corpus_, batch_inputs, entity_embeddings, relation_embed, edge_list, edge_type, edge_embed, edge_list_nhop, edge_type_nhop, a_heads, a2_heads, W, a_out, a2_out):
    raise NotImplementedError("write your pallas kernel here")


# speedup vs baseline: 2.6571x; 2.6571x over previous
import functools
import jax
import jax.numpy as jnp
from jax import lax
from jax.experimental import pallas as pl
from jax.experimental.pallas import tpu as pltpu
from jax.experimental.pallas import tpu_sc as plsc

N = 10000
E1 = 160000
E2 = 80000
NC, NS, L = 2, 16, 16
NW = NC * NS
CH = 128
NCH1 = E1 // CH
NCH2 = E2 // CH
NPAD = 10240
RPT = NPAD // NS
WD = 80


def _elu(v):
    return jnp.where(v > 0, v, jnp.exp(v) - 1.0)


def _mm_body(x_ref, w_ref, o_ref):
    o_ref[...] = jnp.dot(x_ref[...], w_ref[...],
                         preferred_element_type=jnp.float32)


def _matmul(x, w, bm):
    M, K = x.shape
    Nn = w.shape[1]
    return pl.pallas_call(
        _mm_body,
        grid=(M // bm,),
        in_specs=[pl.BlockSpec((bm, K), lambda i: (i, 0)),
                  pl.BlockSpec((K, Nn), lambda i: (0, 0))],
        out_specs=pl.BlockSpec((bm, Nn), lambda i: (i, 0)),
        out_shape=jax.ShapeDtypeStruct((M, Nn), jnp.float32),
    )(x, w)


def _mid_body(acc_ref, bsd_ref, q_ref):
    A = acc_ref[0] + acc_ref[1]
    rs0 = A[:, 64:65]
    rs1 = A[:, 65:66]
    rs0 = jnp.where(rs0 == 0.0, 1e-12, rs0)
    rs1 = jnp.where(rs1 == 0.0, 1e-12, rs1)
    x1 = jnp.concatenate(
        [_elu(A[:, 0:32] / rs0), _elu(A[:, 32:64] / rs1)], axis=1)
    q_ref[...] = jnp.dot(x1, bsd_ref[...], preferred_element_type=jnp.float32)


def _fin_body(acc_ref, o_ref):
    A = acc_ref[0] + acc_ref[1]
    rs = jnp.where(A[:, 64:65] == 0.0, 1e-12, A[:, 64:65])
    o_ref[...] = _elu(A[:, 0:64] / rs)


_MESH = plsc.VectorSubcoreMesh(core_axis_name="c", subcore_axis_name="s")


def _zero_acc(zeros_hbm, acc, sid):
    pltpu.sync_copy(zeros_hbm, acc.at[pl.ds(sid * RPT, RPT)])


def _copy_out(acc, out_hbm, cid, sid):
    pltpu.sync_copy(acc.at[pl.ds(sid * RPT, RPT)],
                    out_hbm.at[cid, pl.ds(sid * RPT, RPT)])


def _chunk_compute(bufs, wrows, layer2, xl_rows, xw_rows):
    lane = jnp.arange(L, dtype=jnp.int32)

    def rsum(i, q):
        v = bufs[0][i, pl.ds(q * L, L)]
        for b in bufs[1:]:
            v = v + b[i, pl.ds(q * L, L)]
        return v

    def body(i, _):
        s = rsum(i, 4)
        wv = jnp.exp(-jnp.where(s >= 0, s, 0.2 * s))
        wrows[i, pl.ds(4 * L, L)] = jnp.where(lane < (1 if layer2 else 2),
                                              wv, 0.0)
        w0 = wv[0]
        w1 = w0 if layer2 else wv[1]
        for q in range(4):
            v = rsum(i, q)
            wm = v * (w0 if q < 2 else w1)
            wrows[i, pl.ds(q * L, L)] = wm
            if layer2:
                xl_rows[i, pl.ds(q * L, L)] = _elu(v)
                xw_rows[i, pl.ds(q * L, L)] = _elu(wm)
        return 0

    lax.fori_loop(0, CH, body, 0)


@functools.partial(
    pl.kernel,
    out_type=[jax.ShapeDtypeStruct((NC, NPAD, WD), jnp.float32)],
    mesh=_MESH,
    compiler_params=pltpu.CompilerParams(use_tc_tiling_on_sc=False),
    scratch_types=[
        pltpu.VMEM_SHARED((NPAD, WD), jnp.float32),
        pltpu.VMEM((CH, WD), jnp.float32),
        pltpu.VMEM((CH, WD), jnp.float32),
        pltpu.VMEM((CH, WD), jnp.float32),
        pltpu.VMEM((CH, WD), jnp.float32),
        pltpu.VMEM((CH, WD), jnp.float32),
        pltpu.VMEM((CH,), jnp.int32),
        pltpu.VMEM((CH,), jnp.int32),
        pltpu.VMEM((CH,), jnp.int32),
        pltpu.VMEM((CH,), jnp.int32),
    ],
)
def _l1_kernel(s_hbm, d_hbm, t0_hbm, t1_hbm, re_hbm, ps_hbm, pd_hbm,
               arel_hbm, zeros_hbm, out_hbm,
               acc, m80, ps_rows, pd_rows, t1_rows, wrows,
               sidx, didx, t0i, t1i):
    cid = lax.axis_index("c")
    sid = lax.axis_index("s")
    wid = sid * NC + cid
    _zero_acc(zeros_hbm, acc, sid)
    plsc.subcore_barrier()

    n1 = (NCH1 - wid + NW - 1) // NW

    def chunk1(k, _):
        base = (wid + k * NW) * CH
        pltpu.sync_copy(s_hbm.at[pl.ds(base, CH)], sidx)
        pltpu.sync_copy(d_hbm.at[pl.ds(base, CH)], didx)
        pltpu.sync_copy(re_hbm.at[pl.ds(base, CH)], m80)
        pltpu.sync_copy(ps_hbm.at[sidx], ps_rows)
        pltpu.sync_copy(pd_hbm.at[didx], pd_rows)
        _chunk_compute([m80, ps_rows, pd_rows], wrows, False, None, None)
        pltpu.sync_copy(wrows, acc.at[sidx], add=True)
        return 0

    lax.fori_loop(0, n1, chunk1, 0)

    n2 = (NCH2 - wid + NW - 1) // NW

    def chunk2(k, _):
        base = (wid + k * NW) * CH
        pltpu.sync_copy(s_hbm.at[pl.ds(E1 + base, CH)], sidx)
        pltpu.sync_copy(d_hbm.at[pl.ds(E1 + base, CH)], didx)
        pltpu.sync_copy(t0_hbm.at[pl.ds(base, CH)], t0i)
        pltpu.sync_copy(t1_hbm.at[pl.ds(base, CH)], t1i)
        pltpu.sync_copy(arel_hbm.at[t0i], m80)
        pltpu.sync_copy(arel_hbm.at[t1i], t1_rows)
        pltpu.sync_copy(ps_hbm.at[sidx], ps_rows)
        pltpu.sync_copy(pd_hbm.at[didx], pd_rows)
        _chunk_compute([m80, t1_rows, ps_rows, pd_rows], wrows, False,
                       None, None)
        pltpu.sync_copy(wrows, acc.at[sidx], add=True)
        return 0

    lax.fori_loop(0, n2, chunk2, 0)
    plsc.subcore_barrier()
    _copy_out(acc, out_hbm, cid, sid)


@functools.partial(
    pl.kernel,
    out_type=[jax.ShapeDtypeStruct((NC, NPAD, WD), jnp.float32),
              jax.ShapeDtypeStruct((E1 + E2, 64), jnp.float32),
              jax.ShapeDtypeStruct((E1 + E2, 64), jnp.float32)],
    mesh=_MESH,
    compiler_params=pltpu.CompilerParams(use_tc_tiling_on_sc=False),
    scratch_types=[
        pltpu.VMEM_SHARED((NPAD, WD), jnp.float32),
        pltpu.VMEM((CH, WD), jnp.float32),
        pltpu.VMEM((CH, WD), jnp.float32),
        pltpu.VMEM((CH, WD), jnp.float32),
        pltpu.VMEM((CH, WD), jnp.float32),
        pltpu.VMEM((CH, WD), jnp.float32),
        pltpu.VMEM((CH, 64), jnp.float32),
        pltpu.VMEM((CH, 64), jnp.float32),
        pltpu.VMEM((CH,), jnp.int32),
        pltpu.VMEM((CH,), jnp.int32),
        pltpu.VMEM((CH,), jnp.int32),
        pltpu.VMEM((CH,), jnp.int32),
    ],
)
def _l2_kernel(s_hbm, d_hbm, et_hbm, t0_hbm, t1_hbm, qs_hbm, qd_hbm,
               w2_hbm, zeros_hbm, out_hbm, xl_hbm, xw_hbm,
               acc, m80, qs_rows, qd_rows, t1_rows, wrows, xl_rows, xw_rows,
               sidx, didx, t0i, t1i):
    cid = lax.axis_index("c")
    sid = lax.axis_index("s")
    wid = sid * NC + cid
    _zero_acc(zeros_hbm, acc, sid)
    plsc.subcore_barrier()

    n1 = (NCH1 - wid + NW - 1) // NW

    def chunk1(k, _):
        base = (wid + k * NW) * CH
        pltpu.sync_copy(s_hbm.at[pl.ds(base, CH)], sidx)
        pltpu.sync_copy(d_hbm.at[pl.ds(base, CH)], didx)
        pltpu.sync_copy(et_hbm.at[pl.ds(base, CH)], t0i)
        pltpu.sync_copy(w2_hbm.at[t0i], m80)
        pltpu.sync_copy(qs_hbm.at[sidx], qs_rows)
        pltpu.sync_copy(qd_hbm.at[didx], qd_rows)
        _chunk_compute([m80, qs_rows, qd_rows], wrows, True, xl_rows, xw_rows)
        pltpu.sync_copy(xl_rows, xl_hbm.at[pl.ds(base, CH)])
        pltpu.sync_copy(xw_rows, xw_hbm.at[pl.ds(base, CH)])
        pltpu.sync_copy(wrows, acc.at[sidx], add=True)
        return 0

    lax.fori_loop(0, n1, chunk1, 0)

    n2 = (NCH2 - wid + NW - 1) // NW

    def chunk2(k, _):
        base = (wid + k * NW) * CH
        pltpu.sync_copy(s_hbm.at[pl.ds(E1 + base, CH)], sidx)
        pltpu.sync_copy(d_hbm.at[pl.ds(E1 + base, CH)], didx)
        pltpu.sync_copy(t0_hbm.at[pl.ds(base, CH)], t0i)
        pltpu.sync_copy(t1_hbm.at[pl.ds(base, CH)], t1i)
        pltpu.sync_copy(w2_hbm.at[t0i], m80)
        pltpu.sync_copy(w2_hbm.at[t1i], t1_rows)
        pltpu.sync_copy(qs_hbm.at[sidx], qs_rows)
        pltpu.sync_copy(qd_hbm.at[didx], qd_rows)
        _chunk_compute([m80, t1_rows, qs_rows, qd_rows], wrows, True,
                       xl_rows, xw_rows)
        pltpu.sync_copy(xl_rows, xl_hbm.at[pl.ds(E1 + base, CH)])
        pltpu.sync_copy(xw_rows, xw_hbm.at[pl.ds(E1 + base, CH)])
        pltpu.sync_copy(wrows, acc.at[sidx], add=True)
        return 0

    lax.fori_loop(0, n2, chunk2, 0)
    plsc.subcore_barrier()
    _copy_out(acc, out_hbm, cid, sid)


def kernel(Corpus_, batch_inputs, entity_embeddings, relation_embed,
           edge_list, edge_type, edge_embed, edge_list_nhop, edge_type_nhop,
           a_heads, a2_heads, W, a_out, a2_out):
    f32 = jnp.float32
    x = entity_embeddings.astype(f32)
    rel = relation_embed.astype(f32)

    a0, a1 = a_heads[0], a_heads[1]
    As = jnp.concatenate([a0[:, :128], a1[:, :128]], 0)
    Ad = jnp.concatenate([a0[:, 128:256], a1[:, 128:256]], 0)
    Ar = jnp.concatenate([a0[:, 256:], a1[:, 256:]], 0)
    a2_0 = a2_heads[0, 0]
    a2_1 = a2_heads[1, 0]
    z14 = jnp.zeros((1, 14), f32)

    def widen(P):
        K = P.shape[0]
        return jnp.concatenate(
            [P, (P[:, :32] @ a2_0)[:, None], (P[:, 32:] @ a2_1)[:, None],
             jnp.broadcast_to(z14, (K, 14))], axis=1)

    As80 = widen(As.T)
    Ad80 = widen(Ad.T)
    Ar80 = widen(Ar.T)

    Bs_t = a_out[:, 0:64].T
    Bd_t = a_out[:, 64:128].T
    Br_t = a_out[:, 128:192].T
    a2o = a2_out[0]
    z15 = jnp.zeros((1, 15), f32)

    def widen2(P):
        K = P.shape[0]
        return jnp.concatenate(
            [P, (P @ a2o)[:, None], jnp.broadcast_to(z15, (K, 15))], axis=1)

    Bs80 = widen2(Bs_t)
    Bd80 = widen2(Bd_t)
    Br80 = widen2(Br_t)
    Bsd160 = jnp.concatenate([Bs80, Bd80], axis=1)

    i32 = jnp.int32
    s_all = jnp.concatenate([edge_list[0], edge_list_nhop[0]]).astype(i32)
    d_all = jnp.concatenate([edge_list[1], edge_list_nhop[1]]).astype(i32)
    et = edge_type.astype(i32)
    t0 = edge_type_nhop[:, 0].astype(i32)
    t1 = edge_type_nhop[:, 1].astype(i32)
    zeros_rp = jnp.zeros((RPT, WD), f32)

    X160 = _matmul(x, jnp.concatenate([As80, Ad80], axis=1), 2000)
    PS80 = X160[:, :80]
    PD80 = X160[:, 80:]
    RE80 = _matmul(edge_embed.astype(f32), Ar80, 8000)
    Y144 = _matmul(rel, jnp.concatenate([Ar80, W.astype(f32)], axis=1), 500)
    AREL80 = Y144[:, :80]
    out_relation_1 = Y144[:, 80:]
    W280 = _matmul(out_relation_1, Br80, 500)

    (acc1,) = _l1_kernel(s_all, d_all, t0, t1, RE80, PS80, PD80,
                         AREL80, zeros_rp)

    Q160 = pl.pallas_call(
        _mid_body,
        grid=(5,),
        in_specs=[pl.BlockSpec((NC, 2048, WD), lambda i: (0, i, 0)),
                  pl.BlockSpec((64, 160), lambda i: (0, 0))],
        out_specs=pl.BlockSpec((2048, 160), lambda i: (i, 0)),
        out_shape=jax.ShapeDtypeStruct((NPAD, 160), f32),
    )(acc1, Bsd160)
    QS80 = Q160[:, :80]
    QD80 = Q160[:, 80:]

    acc2, xl, xw = _l2_kernel(s_all, d_all, et, t0, t1, QS80, QD80,
                              W280, zeros_rp)

    x2 = pl.pallas_call(
        _fin_body,
        grid=(5,),
        in_specs=[pl.BlockSpec((NC, 2048, WD), lambda i: (0, i, 0))],
        out_specs=pl.BlockSpec((2048, 64), lambda i: (i, 0)),
        out_shape=jax.ShapeDtypeStruct((NPAD, 64), f32),
    )(acc2)

    return (x2[:N], out_relation_1, xl, xw)


# speedup vs baseline: 3.0539x; 1.1493x over previous
import functools
import jax
import jax.numpy as jnp
from jax import lax
from jax.experimental import pallas as pl
from jax.experimental.pallas import tpu as pltpu
from jax.experimental.pallas import tpu_sc as plsc

N = 10000
E1 = 160000
E2 = 80000
NC, NS, L = 2, 16, 16
NW = NC * NS
CH = 128
NCH1 = E1 // CH
NCH2 = E2 // CH
NPAD = 10240
RPT = NPAD // NS
WD = 80


def _elu(v):
    return jnp.where(v > 0, v, jnp.exp(v) - 1.0)


def _mm2_body(split, x_ref, w_ref, o1_ref, o2_ref):
    r = jnp.dot(x_ref[...], w_ref[...], preferred_element_type=jnp.float32)
    o1_ref[...] = r[:, :split]
    o2_ref[...] = r[:, split:]


def _matmul2(x, w, bm, split):
    M, K = x.shape
    Nn = w.shape[1]
    return pl.pallas_call(
        functools.partial(_mm2_body, split),
        grid=(M // bm,),
        in_specs=[pl.BlockSpec((bm, K), lambda i: (i, 0)),
                  pl.BlockSpec((K, Nn), lambda i: (0, 0))],
        out_specs=[pl.BlockSpec((bm, split), lambda i: (i, 0)),
                   pl.BlockSpec((bm, Nn - split), lambda i: (i, 0))],
        out_shape=[jax.ShapeDtypeStruct((M, split), jnp.float32),
                   jax.ShapeDtypeStruct((M, Nn - split), jnp.float32)],
    )(x, w)


def _mm_body(x_ref, w_ref, o_ref):
    o_ref[...] = jnp.dot(x_ref[...], w_ref[...],
                         preferred_element_type=jnp.float32)


def _matmul(x, w, bm):
    M, K = x.shape
    Nn = w.shape[1]
    return pl.pallas_call(
        _mm_body,
        grid=(M // bm,),
        in_specs=[pl.BlockSpec((bm, K), lambda i: (i, 0)),
                  pl.BlockSpec((K, Nn), lambda i: (0, 0))],
        out_specs=pl.BlockSpec((bm, Nn), lambda i: (i, 0)),
        out_shape=jax.ShapeDtypeStruct((M, Nn), jnp.float32),
    )(x, w)


def _mid_body(acc_ref, bsd_ref, qs_ref, qd_ref):
    A = acc_ref[0] + acc_ref[1]
    rs0 = A[:, 64:65]
    rs1 = A[:, 65:66]
    rs0 = jnp.where(rs0 == 0.0, 1e-12, rs0)
    rs1 = jnp.where(rs1 == 0.0, 1e-12, rs1)
    x1 = jnp.concatenate(
        [_elu(A[:, 0:32] / rs0), _elu(A[:, 32:64] / rs1)], axis=1)
    q = jnp.dot(x1, bsd_ref[...], preferred_element_type=jnp.float32)
    qs_ref[...] = q[:, :80]
    qd_ref[...] = q[:, 80:]


def _fin_body(acc_ref, o_ref):
    A = acc_ref[0] + acc_ref[1]
    rs = jnp.where(A[:, 64:65] == 0.0, 1e-12, A[:, 64:65])
    o_ref[...] = _elu(A[:, 0:64] / rs)


_MESH = plsc.VectorSubcoreMesh(core_axis_name="c", subcore_axis_name="s")


def _zero_acc(zeros_hbm, acc, sid):
    pltpu.sync_copy(zeros_hbm, acc.at[pl.ds(sid * RPT, RPT)])


def _copy_out(acc, out_hbm, cid, sid):
    pltpu.sync_copy(acc.at[pl.ds(sid * RPT, RPT)],
                    out_hbm.at[cid, pl.ds(sid * RPT, RPT)])


def _chunk_compute(bufs, wrows, layer2, xl_rows, xw_rows):
    lane = jnp.arange(L, dtype=jnp.int32)

    def rsum(i, q):
        v = bufs[0][i, pl.ds(q * L, L)]
        for b in bufs[1:]:
            v = v + b[i, pl.ds(q * L, L)]
        return v

    def body(i, _):
        s = rsum(i, 4)
        wv = jnp.exp(-jnp.where(s >= 0, s, 0.2 * s))
        wrows[i, pl.ds(4 * L, L)] = jnp.where(lane < (1 if layer2 else 2),
                                              wv, 0.0)
        w0 = wv[0]
        w1 = w0 if layer2 else wv[1]
        for q in range(4):
            v = rsum(i, q)
            wm = v * (w0 if q < 2 else w1)
            wrows[i, pl.ds(q * L, L)] = wm
            if layer2:
                xl_rows[i, pl.ds(q * L, L)] = _elu(v)
                xw_rows[i, pl.ds(q * L, L)] = _elu(wm)
        return 0

    lax.fori_loop(0, CH, body, 0)


@functools.partial(
    pl.kernel,
    out_type=[jax.ShapeDtypeStruct((NC, NPAD, WD), jnp.float32)],
    mesh=_MESH,
    compiler_params=pltpu.CompilerParams(use_tc_tiling_on_sc=False),
    scratch_types=[
        pltpu.VMEM_SHARED((NPAD, WD), jnp.float32),
        pltpu.VMEM((CH, WD), jnp.float32),
        pltpu.VMEM((CH, WD), jnp.float32),
        pltpu.VMEM((CH, WD), jnp.float32),
        pltpu.VMEM((CH, WD), jnp.float32),
        pltpu.VMEM((CH, WD), jnp.float32),
        pltpu.VMEM((CH,), jnp.int32),
        pltpu.VMEM((CH,), jnp.int32),
        pltpu.VMEM((CH,), jnp.int32),
        pltpu.VMEM((CH,), jnp.int32),
        pltpu.SemaphoreType.DMA,
    ],
)
def _l1_kernel(s_hbm, d_hbm, t0_hbm, t1_hbm, re_hbm, ps_hbm, pd_hbm,
               arel_hbm, zeros_hbm, out_hbm,
               acc, m80, ps_rows, pd_rows, t1_rows, wrows,
               sidx, didx, t0i, t1i, sem):
    cid = lax.axis_index("c")
    sid = lax.axis_index("s")
    wid = sid * NC + cid
    _zero_acc(zeros_hbm, acc, sid)
    plsc.subcore_barrier()

    n1 = (NCH1 - wid + NW - 1) // NW

    def chunk1(k, _):
        base = (wid + k * NW) * CH
        i1 = pltpu.async_copy(s_hbm.at[pl.ds(base, CH)], sidx, sem)
        i2 = pltpu.async_copy(d_hbm.at[pl.ds(base, CH)], didx, sem)
        i1.wait()
        i2.wait()
        c1 = pltpu.async_copy(re_hbm.at[pl.ds(base, CH)], m80, sem)
        c2 = pltpu.async_copy(ps_hbm.at[sidx], ps_rows, sem)
        c3 = pltpu.async_copy(pd_hbm.at[didx], pd_rows, sem)
        c1.wait()
        c2.wait()
        c3.wait()
        _chunk_compute([m80, ps_rows, pd_rows], wrows, False, None, None)
        pltpu.sync_copy(wrows, acc.at[sidx], add=True)
        return 0

    lax.fori_loop(0, n1, chunk1, 0)

    n2 = (NCH2 - wid + NW - 1) // NW

    def chunk2(k, _):
        base = (wid + k * NW) * CH
        i1 = pltpu.async_copy(s_hbm.at[pl.ds(E1 + base, CH)], sidx, sem)
        i2 = pltpu.async_copy(d_hbm.at[pl.ds(E1 + base, CH)], didx, sem)
        i3 = pltpu.async_copy(t0_hbm.at[pl.ds(base, CH)], t0i, sem)
        i4 = pltpu.async_copy(t1_hbm.at[pl.ds(base, CH)], t1i, sem)
        i1.wait()
        i2.wait()
        i3.wait()
        i4.wait()
        c1 = pltpu.async_copy(arel_hbm.at[t0i], m80, sem)
        c2 = pltpu.async_copy(arel_hbm.at[t1i], t1_rows, sem)
        c3 = pltpu.async_copy(ps_hbm.at[sidx], ps_rows, sem)
        c4 = pltpu.async_copy(pd_hbm.at[didx], pd_rows, sem)
        c1.wait()
        c2.wait()
        c3.wait()
        c4.wait()
        _chunk_compute([m80, t1_rows, ps_rows, pd_rows], wrows, False,
                       None, None)
        pltpu.sync_copy(wrows, acc.at[sidx], add=True)
        return 0

    lax.fori_loop(0, n2, chunk2, 0)
    plsc.subcore_barrier()
    _copy_out(acc, out_hbm, cid, sid)


@functools.partial(
    pl.kernel,
    out_type=[jax.ShapeDtypeStruct((NC, NPAD, WD), jnp.float32),
              jax.ShapeDtypeStruct((E1 + E2, 64), jnp.float32),
              jax.ShapeDtypeStruct((E1 + E2, 64), jnp.float32)],
    mesh=_MESH,
    compiler_params=pltpu.CompilerParams(use_tc_tiling_on_sc=False),
    scratch_types=[
        pltpu.VMEM_SHARED((NPAD, WD), jnp.float32),
        pltpu.VMEM((CH, WD), jnp.float32),
        pltpu.VMEM((CH, WD), jnp.float32),
        pltpu.VMEM((CH, WD), jnp.float32),
        pltpu.VMEM((CH, WD), jnp.float32),
        pltpu.VMEM((CH, WD), jnp.float32),
        pltpu.VMEM((CH, 64), jnp.float32),
        pltpu.VMEM((CH, 64), jnp.float32),
        pltpu.VMEM((CH,), jnp.int32),
        pltpu.VMEM((CH,), jnp.int32),
        pltpu.VMEM((CH,), jnp.int32),
        pltpu.VMEM((CH,), jnp.int32),
        pltpu.SemaphoreType.DMA,
    ],
)
def _l2_kernel(s_hbm, d_hbm, et_hbm, t0_hbm, t1_hbm, qs_hbm, qd_hbm,
               w2_hbm, zeros_hbm, out_hbm, xl_hbm, xw_hbm,
               acc, m80, qs_rows, qd_rows, t1_rows, wrows, xl_rows, xw_rows,
               sidx, didx, t0i, t1i, sem):
    cid = lax.axis_index("c")
    sid = lax.axis_index("s")
    wid = sid * NC + cid
    _zero_acc(zeros_hbm, acc, sid)
    plsc.subcore_barrier()

    n1 = (NCH1 - wid + NW - 1) // NW

    def chunk1(k, _):
        base = (wid + k * NW) * CH
        i1 = pltpu.async_copy(s_hbm.at[pl.ds(base, CH)], sidx, sem)
        i2 = pltpu.async_copy(d_hbm.at[pl.ds(base, CH)], didx, sem)
        i3 = pltpu.async_copy(et_hbm.at[pl.ds(base, CH)], t0i, sem)
        i1.wait()
        i2.wait()
        i3.wait()
        c1 = pltpu.async_copy(w2_hbm.at[t0i], m80, sem)
        c2 = pltpu.async_copy(qs_hbm.at[sidx], qs_rows, sem)
        c3 = pltpu.async_copy(qd_hbm.at[didx], qd_rows, sem)
        c1.wait()
        c2.wait()
        c3.wait()
        _chunk_compute([m80, qs_rows, qd_rows], wrows, True, xl_rows, xw_rows)
        pltpu.sync_copy(xl_rows, xl_hbm.at[pl.ds(base, CH)])
        pltpu.sync_copy(xw_rows, xw_hbm.at[pl.ds(base, CH)])
        pltpu.sync_copy(wrows, acc.at[sidx], add=True)
        return 0

    lax.fori_loop(0, n1, chunk1, 0)

    n2 = (NCH2 - wid + NW - 1) // NW

    def chunk2(k, _):
        base = (wid + k * NW) * CH
        i1 = pltpu.async_copy(s_hbm.at[pl.ds(E1 + base, CH)], sidx, sem)
        i2 = pltpu.async_copy(d_hbm.at[pl.ds(E1 + base, CH)], didx, sem)
        i3 = pltpu.async_copy(t0_hbm.at[pl.ds(base, CH)], t0i, sem)
        i4 = pltpu.async_copy(t1_hbm.at[pl.ds(base, CH)], t1i, sem)
        i1.wait()
        i2.wait()
        i3.wait()
        i4.wait()
        c1 = pltpu.async_copy(w2_hbm.at[t0i], m80, sem)
        c2 = pltpu.async_copy(w2_hbm.at[t1i], t1_rows, sem)
        c3 = pltpu.async_copy(qs_hbm.at[sidx], qs_rows, sem)
        c4 = pltpu.async_copy(qd_hbm.at[didx], qd_rows, sem)
        c1.wait()
        c2.wait()
        c3.wait()
        c4.wait()
        _chunk_compute([m80, t1_rows, qs_rows, qd_rows], wrows, True,
                       xl_rows, xw_rows)
        pltpu.sync_copy(xl_rows, xl_hbm.at[pl.ds(E1 + base, CH)])
        pltpu.sync_copy(xw_rows, xw_hbm.at[pl.ds(E1 + base, CH)])
        pltpu.sync_copy(wrows, acc.at[sidx], add=True)
        return 0

    lax.fori_loop(0, n2, chunk2, 0)
    plsc.subcore_barrier()
    _copy_out(acc, out_hbm, cid, sid)


def kernel(Corpus_, batch_inputs, entity_embeddings, relation_embed,
           edge_list, edge_type, edge_embed, edge_list_nhop, edge_type_nhop,
           a_heads, a2_heads, W, a_out, a2_out):
    f32 = jnp.float32
    x = entity_embeddings.astype(f32)
    rel = relation_embed.astype(f32)

    a0, a1 = a_heads[0], a_heads[1]
    As = jnp.concatenate([a0[:, :128], a1[:, :128]], 0)
    Ad = jnp.concatenate([a0[:, 128:256], a1[:, 128:256]], 0)
    Ar = jnp.concatenate([a0[:, 256:], a1[:, 256:]], 0)
    a2_0 = a2_heads[0, 0]
    a2_1 = a2_heads[1, 0]
    z14 = jnp.zeros((1, 14), f32)

    def widen(P):
        K = P.shape[0]
        return jnp.concatenate(
            [P, (P[:, :32] @ a2_0)[:, None], (P[:, 32:] @ a2_1)[:, None],
             jnp.broadcast_to(z14, (K, 14))], axis=1)

    As80 = widen(As.T)
    Ad80 = widen(Ad.T)
    Ar80 = widen(Ar.T)

    Bs_t = a_out[:, 0:64].T
    Bd_t = a_out[:, 64:128].T
    Br_t = a_out[:, 128:192].T
    a2o = a2_out[0]
    z15 = jnp.zeros((1, 15), f32)

    def widen2(P):
        K = P.shape[0]
        return jnp.concatenate(
            [P, (P @ a2o)[:, None], jnp.broadcast_to(z15, (K, 15))], axis=1)

    Bs80 = widen2(Bs_t)
    Bd80 = widen2(Bd_t)
    Br80 = widen2(Br_t)
    Bsd160 = jnp.concatenate([Bs80, Bd80], axis=1)

    i32 = jnp.int32
    s_all = jnp.concatenate([edge_list[0], edge_list_nhop[0]]).astype(i32)
    d_all = jnp.concatenate([edge_list[1], edge_list_nhop[1]]).astype(i32)
    et = edge_type.astype(i32)
    t0 = edge_type_nhop[:, 0].astype(i32)
    t1 = edge_type_nhop[:, 1].astype(i32)
    zeros_rp = jnp.zeros((RPT, WD), f32)

    PS80, PD80 = _matmul2(x, jnp.concatenate([As80, Ad80], axis=1), 2000, 80)
    RE80 = _matmul(edge_embed.astype(f32), Ar80, 8000)
    AREL80, out_relation_1 = _matmul2(
        rel, jnp.concatenate([Ar80, W.astype(f32)], axis=1), 500, 80)
    W280 = _matmul(out_relation_1, Br80, 500)

    (acc1,) = _l1_kernel(s_all, d_all, t0, t1, RE80, PS80, PD80,
                         AREL80, zeros_rp)

    QS80, QD80 = pl.pallas_call(
        _mid_body,
        grid=(5,),
        in_specs=[pl.BlockSpec((NC, 2048, WD), lambda i: (0, i, 0)),
                  pl.BlockSpec((64, 160), lambda i: (0, 0))],
        out_specs=[pl.BlockSpec((2048, 80), lambda i: (i, 0)),
                   pl.BlockSpec((2048, 80), lambda i: (i, 0))],
        out_shape=[jax.ShapeDtypeStruct((NPAD, 80), f32),
                   jax.ShapeDtypeStruct((NPAD, 80), f32)],
    )(acc1, Bsd160)

    acc2, xl, xw = _l2_kernel(s_all, d_all, et, t0, t1, QS80, QD80,
                              W280, zeros_rp)

    x2 = pl.pallas_call(
        _fin_body,
        grid=(5,),
        in_specs=[pl.BlockSpec((NC, 2048, WD), lambda i: (0, i, 0))],
        out_specs=pl.BlockSpec((2048, 64), lambda i: (i, 0)),
        out_shape=jax.ShapeDtypeStruct((NPAD, 64), f32),
    )(acc2)

    return (x2[:N], out_relation_1, xl, xw)
